# baseline (speedup 1.0000x reference)
import jax
import jax.numpy as jnp
from jax.experimental import pallas as pl
from jax.experimental.pallas import tpu as pltpu

_BS = 256


def _body(hs_ref, p_ref, m_ref, out_ref, tpe_ref):
    m = m_ref[...]
    w = jnp.sum(p_ref[...] * m, axis=1)
    bs, b, _ = hs_ref.shape
    out_ref[...] = hs_ref[...] * w.reshape(bs, b, 1)

    @pl.when(pl.program_id(0) == 0)
    def _init():
        tpe_ref[...] = jnp.zeros_like(tpe_ref)

    tpe_ref[...] += jnp.sum(m, axis=0, keepdims=True)


def kernel(hidden_states, probs, routing_map):
    S, B, H = hidden_states.shape
    T, E = probs.shape
    mask = routing_map.astype(jnp.float32)

    grid = (S // _BS,)
    out, tpe = pl.pallas_call(
        _body,
        grid=grid,
        in_specs=[
            pl.BlockSpec((_BS, B, H), lambda i: (i, 0, 0)),
            pl.BlockSpec((_BS * B, E), lambda i: (i, 0)),
            pl.BlockSpec((_BS * B, E), lambda i: (i, 0)),
        ],
        out_specs=[
            pl.BlockSpec((_BS, B, H), lambda i: (i, 0, 0)),
            pl.BlockSpec((1, E), lambda i: (0, 0)),
        ],
        out_shape=[
            jax.ShapeDtypeStruct((S, B, H), hidden_states.dtype),
            jax.ShapeDtypeStruct((1, E), jnp.float32),
        ],
    )(hidden_states, probs, mask)

    tokens_per_expert = tpe.reshape(E).astype(jnp.int32)
    return out, tokens_per_expert


# speedup vs baseline: 1.1412x; 1.1412x over previous
import jax
import jax.numpy as jnp
from jax.experimental import pallas as pl
from jax.experimental.pallas import tpu as pltpu

_BS = 1024


def _body(hs_ref, p_ref, m_ref, out_ref, tpe_ref):
    m = m_ref[...]
    w = jnp.sum(p_ref[...] * m, axis=1)
    bs, b, _ = hs_ref.shape
    out_ref[...] = hs_ref[...] * w.reshape(bs, b, 1)

    @pl.when(pl.program_id(0) == 0)
    def _init():
        tpe_ref[...] = jnp.zeros_like(tpe_ref)

    tpe_ref[...] += jnp.sum(m, axis=0, keepdims=True)


def kernel(hidden_states, probs, routing_map):
    S, B, H = hidden_states.shape
    T, E = probs.shape
    mask = routing_map.astype(jnp.float32)

    grid = (S // _BS,)
    out, tpe = pl.pallas_call(
        _body,
        grid=grid,
        in_specs=[
            pl.BlockSpec((_BS, B, H), lambda i: (i, 0, 0)),
            pl.BlockSpec((_BS * B, E), lambda i: (i, 0)),
            pl.BlockSpec((_BS * B, E), lambda i: (i, 0)),
        ],
        out_specs=[
            pl.BlockSpec((_BS, B, H), lambda i: (i, 0, 0)),
            pl.BlockSpec((1, E), lambda i: (0, 0)),
        ],
        out_shape=[
            jax.ShapeDtypeStruct((S, B, H), hidden_states.dtype),
            jax.ShapeDtypeStruct((1, E), jnp.float32),
        ],
    )(hidden_states, probs, mask)

    tokens_per_expert = tpe.reshape(E).astype(jnp.int32)
    return out, tokens_per_expert
